# Initial kernel scaffold; baseline (speedup 1.0000x reference)
import functools

import jax
import jax.numpy as jnp
from jax import lax
from jax.experimental import pallas as pl
from jax.experimental.pallas import tpu as pltpu
from jax.experimental.pallas import tpu_sc as plsc

N = 10000
E = 320000
D_IN = 128
HID = 16
HEADS = 8
D_OUT = 64

NPAD = 10240
NW = 32
K = 128
EW = 10112
EPAD = NW * EW
NCHUNK = EW // K
ROWS_PER_TILE = NPAD // 16

ACC1_W = 144
ACC2_W = 80

F32 = jnp.float32
I32 = jnp.int32


def _tc_a_body(x_ref, w1_ref, af_ref, df_ref, gdup_ref, h_ref, as_ref, ad_ref):
    x = x_ref[...]
    h = jnp.dot(x, w1_ref[...], preferred_element_type=F32)
    h_ref[...] = h
    gdup = gdup_ref[...]
    as_ref[...] = jnp.dot(h * af_ref[...], gdup, preferred_element_type=F32)
    ad_ref[...] = jnp.dot(h * df_ref[...], gdup, preferred_element_type=F32)


def _tc_b_body(parts_ref, b1_ref, w2_ref, gexp_ref, a2s_ref, a2d_ref,
               h2_ref, as2_ref, ad2_ref):
    p = parts_ref[0] + parts_ref[1]
    num = p[:, 0:128]
    dend = p[:, 128:144]
    den = jnp.dot(dend, gexp_ref[...], preferred_element_type=F32)
    hcur = jnp.maximum(num / (den + 1e-16) + b1_ref[...], 0.0)
    h2 = jnp.dot(hcur, w2_ref[...], preferred_element_type=F32)
    h2_ref[...] = h2
    as2_ref[...] = jnp.dot(h2, a2s_ref[...], preferred_element_type=F32)
    ad2_ref[...] = jnp.dot(h2, a2d_ref[...], preferred_element_type=F32)


def _tc_c_body(parts_ref, b2_ref, out_ref):
    p = parts_ref[0] + parts_ref[1]
    num = p[:, 0:64]
    den = p[:, 64:65]
    z = num / (den + 1e-16) + b2_ref[...]
    m = jnp.max(z, axis=1, keepdims=True)
    lse = jnp.log(jnp.sum(jnp.exp(z - m), axis=1, keepdims=True)) + m
    out_ref[...] = z - lse


_MESH = plsc.VectorSubcoreMesh(core_axis_name="c", subcore_axis_name="s")


def _make_sc_kernel(feat_w, acc_w, n_head_vec):
    zvecs = acc_w // 16

    @functools.partial(
        pl.kernel,
        out_type=jax.ShapeDtypeStruct((2, NPAD, acc_w), F32),
        mesh=_MESH,
        scratch_types=[
            pltpu.VMEM((K,), I32),
            pltpu.VMEM((K,), I32),
            pltpu.VMEM((K,), F32),
            pltpu.VMEM((K, 16), F32),
            pltpu.VMEM((K, 16), F32),
            pltpu.VMEM((K, feat_w), F32),
            pltpu.VMEM((K, acc_w), F32),
            pltpu.VMEM((K * 16,), F32),
            pltpu.VMEM_SHARED((NPAD, acc_w), F32),
            pltpu.SemaphoreType.DMA,
        ],
    )
    def sc_kernel(sidx, didx, wvals, asrc, adst, htab, out,
                  sidx_v, didx_v, w_v, arows, brows, hrows, msg, coef, acc,
                  sem):
        cid = lax.axis_index("c")
        sid = lax.axis_index("s")
        wid = cid * 16 + sid
        zero16 = jnp.zeros((16,), F32)

        @pl.loop(0, K)
        def _zero_msg(j):
            for t in range(zvecs):
                msg[j, pl.ds(16 * t, 16)] = zero16

        for b in range(ROWS_PER_TILE // K):
            pltpu.sync_copy(msg, acc.at[pl.ds(sid * ROWS_PER_TILE + b * K, K)])
        plsc.subcore_barrier()

        base = wid * EW

        @pl.loop(0, NCHUNK)
        def _chunk(c):
            off = base + c * K
            pltpu.sync_copy(sidx.at[pl.ds(off, K)], sidx_v)
            pltpu.sync_copy(didx.at[pl.ds(off, K)], didx_v)
            pltpu.sync_copy(wvals.at[pl.ds(off, K)], w_v)
            ca = pltpu.async_copy(asrc.at[sidx_v], arows, sem)
            cb = pltpu.async_copy(adst.at[didx_v], brows, sem)
            ch = pltpu.async_copy(htab.at[sidx_v], hrows, sem)
            ca.wait()
            cb.wait()
            ch.wait()

            @pl.loop(0, K)
            def _edge(j):
                va = arows[j, pl.ds(0, 16)]
                vb = brows[j, pl.ds(0, 16)]
                e = va + vb
                e = jnp.where(e >= 0.0, e, 0.2 * e)
                ex = jnp.exp(e)
                wj = plsc.load_gather(w_v, [jnp.full((16,), j, I32)])
                cf = ex * wj
                msg[j, pl.ds(feat_w, 16)] = ex
                if n_head_vec:
                    coef[pl.ds(j * 16, 16)] = cf
                    for h in range(n_head_vec):
                        hv = hrows[j, pl.ds(16 * h, 16)]
                        cs = plsc.load_gather(
                            coef, [jnp.full((16,), j * 16 + h, I32)])
                        msg[j, pl.ds(16 * h, 16)] = hv * cs
                else:
                    for t in range(feat_w // 16):
                        hv = hrows[j, pl.ds(16 * t, 16)]
                        msg[j, pl.ds(16 * t, 16)] = hv * cf

            pltpu.sync_copy(msg, acc.at[didx_v], add=True)

        plsc.subcore_barrier()

        pltpu.sync_copy(
            acc.at[pl.ds(sid * ROWS_PER_TILE, ROWS_PER_TILE)],
            out.at[cid, pl.ds(sid * ROWS_PER_TILE, ROWS_PER_TILE)])

    return sc_kernel


_sc_layer1 = _make_sc_kernel(128, ACC1_W, 8)
_sc_layer2 = _make_sc_kernel(64, ACC2_W, 0)


@jax.jit
def kernel(X, A, W, W1, a_src1, a_dst1, b1, W2, a_src2, a_dst2, b2):
    xp = jnp.zeros((NPAD, D_IN), F32).at[:N].set(X)
    pad_idx = jnp.full((EPAD - E,), NPAD - 1, I32)
    sidx = jnp.concatenate([A[0], pad_idx])
    didx = jnp.concatenate([A[1], pad_idx])
    wv = jnp.concatenate([W, jnp.zeros((EPAD - E,), F32)])

    lane = jnp.arange(128)
    head16 = jnp.arange(16)
    gdup = (lane[:, None] // 16 == (head16[None, :] % 8)).astype(F32)
    gexp = ((head16[:, None] < 8) & (lane[None, :] // 16 == head16[:, None])
            ).astype(F32)
    af = a_src1.reshape(1, 128)
    df = a_dst1.reshape(1, 128)
    a2s = jnp.tile(a_src2.reshape(D_OUT, 1), (1, 16))
    a2d = jnp.tile(a_dst2.reshape(D_OUT, 1), (1, 16))

    blk = 1024
    grid = NPAD // blk

    h1, asd, add_ = pl.pallas_call(
        _tc_a_body,
        grid=(grid,),
        in_specs=[
            pl.BlockSpec((blk, D_IN), lambda i: (i, 0)),
            pl.BlockSpec((D_IN, 128), lambda i: (0, 0)),
            pl.BlockSpec((1, 128), lambda i: (0, 0)),
            pl.BlockSpec((1, 128), lambda i: (0, 0)),
            pl.BlockSpec((128, 16), lambda i: (0, 0)),
        ],
        out_specs=[
            pl.BlockSpec((blk, 128), lambda i: (i, 0)),
            pl.BlockSpec((blk, 16), lambda i: (i, 0)),
            pl.BlockSpec((blk, 16), lambda i: (i, 0)),
        ],
        out_shape=[
            jax.ShapeDtypeStruct((NPAD, 128), F32),
            jax.ShapeDtypeStruct((NPAD, 16), F32),
            jax.ShapeDtypeStruct((NPAD, 16), F32),
        ],
    )(xp, W1, af, df, gdup)

    parts1 = _sc_layer1(sidx, didx, wv, asd, add_, h1)

    h2, as2, ad2 = pl.pallas_call(
        _tc_b_body,
        grid=(grid,),
        in_specs=[
            pl.BlockSpec((2, blk, ACC1_W), lambda i: (0, i, 0)),
            pl.BlockSpec((1, 128), lambda i: (0, 0)),
            pl.BlockSpec((128, D_OUT), lambda i: (0, 0)),
            pl.BlockSpec((16, 128), lambda i: (0, 0)),
            pl.BlockSpec((D_OUT, 16), lambda i: (0, 0)),
            pl.BlockSpec((D_OUT, 16), lambda i: (0, 0)),
        ],
        out_specs=[
            pl.BlockSpec((blk, D_OUT), lambda i: (i, 0)),
            pl.BlockSpec((blk, 16), lambda i: (i, 0)),
            pl.BlockSpec((blk, 16), lambda i: (i, 0)),
        ],
        out_shape=[
            jax.ShapeDtypeStruct((NPAD, D_OUT), F32),
            jax.ShapeDtypeStruct((NPAD, 16), F32),
            jax.ShapeDtypeStruct((NPAD, 16), F32),
        ],
    )(parts1, b1.reshape(1, 128), W2, gexp, a2s, a2d)

    parts2 = _sc_layer2(sidx, didx, wv, as2, ad2, h2)

    out = pl.pallas_call(
        _tc_c_body,
        grid=(grid,),
        in_specs=[
            pl.BlockSpec((2, blk, ACC2_W), lambda i: (0, i, 0)),
            pl.BlockSpec((1, D_OUT), lambda i: (0, 0)),
        ],
        out_specs=pl.BlockSpec((blk, D_OUT), lambda i: (i, 0)),
        out_shape=jax.ShapeDtypeStruct((NPAD, D_OUT), F32),
    )(parts2, b2.reshape(1, D_OUT))

    return out[:N]


# speedup vs baseline: 27.8623x; 27.8623x over previous
import functools

import jax
import jax.numpy as jnp
from jax import lax
from jax.experimental import pallas as pl
from jax.experimental.pallas import tpu as pltpu
from jax.experimental.pallas import tpu_sc as plsc

N = 10000
E = 320000
D_IN = 128
HID = 16
HEADS = 8
D_OUT = 64

NPAD = 10240
NW = 32
K = 128
EW = 10112
EPAD = NW * EW
NCHUNK = EW // K
ROWS_PER_TILE = NPAD // 16

ACC1_W = 136
ACC2_W = 80

F32 = jnp.float32
I32 = jnp.int32


def _tc_a_body(x_ref, w1_ref, af_ref, df_ref, gdup_ref, h_ref, as_ref, ad_ref):
    x = x_ref[...]
    h = jnp.dot(x, w1_ref[...], preferred_element_type=F32)
    h_ref[...] = h
    gdup = gdup_ref[...]
    as_ref[...] = jnp.dot(h * af_ref[...], gdup, preferred_element_type=F32)
    ad_ref[...] = jnp.dot(h * df_ref[...], gdup, preferred_element_type=F32)


def _tc_b_body(parts_ref, b1_ref, w2_ref, gexp_ref, a2s_ref, a2d_ref,
               h2_ref, as2_ref, ad2_ref):
    p = parts_ref[0] + parts_ref[1]
    num = p[:, 0:128]
    dend = p[:, 128:136]
    den = jnp.dot(dend, gexp_ref[...], preferred_element_type=F32)
    hcur = jnp.maximum(num / (den + 1e-16) + b1_ref[...], 0.0)
    h2 = jnp.dot(hcur, w2_ref[...], preferred_element_type=F32)
    h2_ref[...] = h2
    as2_ref[...] = jnp.dot(h2, a2s_ref[...], preferred_element_type=F32)
    ad2_ref[...] = jnp.dot(h2, a2d_ref[...], preferred_element_type=F32)


def _tc_c_body(parts_ref, b2_ref, out_ref):
    p = parts_ref[0] + parts_ref[1]
    num = p[:, 0:64]
    den = p[:, 64:65]
    z = num / (den + 1e-16) + b2_ref[...]
    m = jnp.max(z, axis=1, keepdims=True)
    lse = jnp.log(jnp.sum(jnp.exp(z - m), axis=1, keepdims=True)) + m
    out_ref[...] = z - lse


_MESH = plsc.VectorSubcoreMesh(core_axis_name="c", subcore_axis_name="s")


def _make_sc_kernel(feat_w, acc_w, n_head_vec):
    zvecs = acc_w // 16

    @functools.partial(
        pl.kernel,
        out_type=jax.ShapeDtypeStruct((2, NPAD, acc_w), F32),
        mesh=_MESH,
        compiler_params=pltpu.CompilerParams(
            needs_layout_passes=False, use_tc_tiling_on_sc=False),
        scratch_types=[
            pltpu.VMEM((K,), I32),
            pltpu.VMEM((K,), I32),
            pltpu.VMEM((K,), F32),
            pltpu.VMEM((K, 16), F32),
            pltpu.VMEM((K, 16), F32),
            pltpu.VMEM((K, feat_w), F32),
            pltpu.VMEM((K, acc_w), F32),
            pltpu.VMEM((K * 16,), F32),
            pltpu.VMEM_SHARED((NPAD, acc_w), F32),
            pltpu.SemaphoreType.DMA,
        ],
    )
    def sc_kernel(sidx, didx, wvals, asrc, adst, htab, out,
                  sidx_v, didx_v, w_v, arows, brows, hrows, msg, coef, acc,
                  sem):
        cid = lax.axis_index("c")
        sid = lax.axis_index("s")
        wid = cid * 16 + sid
        zero16 = jnp.zeros((16,), F32)

        @pl.loop(0, K)
        def _zero_msg(j):
            for t in range(zvecs):
                msg[j, pl.ds(16 * t, 16)] = zero16

        for b in range(ROWS_PER_TILE // K):
            pltpu.sync_copy(msg, acc.at[pl.ds(sid * ROWS_PER_TILE + b * K, K)])
        plsc.subcore_barrier()

        base = wid * EW

        @pl.loop(0, NCHUNK)
        def _chunk(c):
            off = base + c * K
            pltpu.sync_copy(sidx.at[pl.ds(off, K)], sidx_v)
            pltpu.sync_copy(didx.at[pl.ds(off, K)], didx_v)
            pltpu.sync_copy(wvals.at[pl.ds(off, K)], w_v)
            ca = pltpu.async_copy(asrc.at[sidx_v], arows, sem)
            cb = pltpu.async_copy(adst.at[didx_v], brows, sem)
            ch = pltpu.async_copy(htab.at[sidx_v], hrows, sem)
            ca.wait()
            cb.wait()
            ch.wait()

            @pl.loop(0, K)
            def _edge(j):
                va = arows[j, pl.ds(0, 16)]
                vb = brows[j, pl.ds(0, 16)]
                e = va + vb
                e = jnp.where(e >= 0.0, e, 0.2 * e)
                ex = jnp.exp(e)
                wj = plsc.load_gather(w_v, [jnp.full((16,), j, I32)])
                cf = ex * wj
                msg[j, pl.ds(feat_w - 8 if n_head_vec else feat_w, 16)] = ex
                if n_head_vec:
                    coef[pl.ds(j * 16, 16)] = cf
                    for h in range(n_head_vec):
                        hv = hrows[j, pl.ds(16 * h, 16)]
                        cs = plsc.load_gather(
                            coef, [jnp.full((16,), j * 16 + h, I32)])
                        msg[j, pl.ds(16 * h, 16)] = hv * cs
                else:
                    for t in range(feat_w // 16):
                        hv = hrows[j, pl.ds(16 * t, 16)]
                        msg[j, pl.ds(16 * t, 16)] = hv * cf

            pltpu.sync_copy(msg, acc.at[didx_v], add=True)

        plsc.subcore_barrier()

        pltpu.sync_copy(
            acc.at[pl.ds(sid * ROWS_PER_TILE, ROWS_PER_TILE)],
            out.at[cid, pl.ds(sid * ROWS_PER_TILE, ROWS_PER_TILE)])

    return sc_kernel


_sc_layer1 = _make_sc_kernel(128, ACC1_W, 8)
_sc_layer2 = _make_sc_kernel(64, ACC2_W, 0)


@jax.jit
def kernel(X, A, W, W1, a_src1, a_dst1, b1, W2, a_src2, a_dst2, b2):
    xp = jnp.zeros((NPAD, D_IN), F32).at[:N].set(X)
    pad_idx = jnp.full((EPAD - E,), NPAD - 1, I32)
    sidx = jnp.concatenate([A[0], pad_idx])
    didx = jnp.concatenate([A[1], pad_idx])
    wv = jnp.concatenate([W, jnp.zeros((EPAD - E,), F32)])

    lane = jnp.arange(128)
    head16 = jnp.arange(16)
    gdup = (lane[:, None] // 16 == (head16[None, :] % 8)).astype(F32)
    head8 = jnp.arange(8)
    gexp = (lane[None, :] // 16 == head8[:, None]).astype(F32)
    af = a_src1.reshape(1, 128)
    df = a_dst1.reshape(1, 128)
    a2s = jnp.tile(a_src2.reshape(D_OUT, 1), (1, 16))
    a2d = jnp.tile(a_dst2.reshape(D_OUT, 1), (1, 16))

    blk = 1024
    grid = NPAD // blk

    h1, asd, add_ = pl.pallas_call(
        _tc_a_body,
        grid=(grid,),
        in_specs=[
            pl.BlockSpec((blk, D_IN), lambda i: (i, 0)),
            pl.BlockSpec((D_IN, 128), lambda i: (0, 0)),
            pl.BlockSpec((1, 128), lambda i: (0, 0)),
            pl.BlockSpec((1, 128), lambda i: (0, 0)),
            pl.BlockSpec((128, 16), lambda i: (0, 0)),
        ],
        out_specs=[
            pl.BlockSpec((blk, 128), lambda i: (i, 0)),
            pl.BlockSpec((blk, 16), lambda i: (i, 0)),
            pl.BlockSpec((blk, 16), lambda i: (i, 0)),
        ],
        out_shape=[
            jax.ShapeDtypeStruct((NPAD, 128), F32),
            jax.ShapeDtypeStruct((NPAD, 16), F32),
            jax.ShapeDtypeStruct((NPAD, 16), F32),
        ],
    )(xp, W1, af, df, gdup)

    parts1 = _sc_layer1(sidx, didx, wv, asd, add_, h1)

    h2, as2, ad2 = pl.pallas_call(
        _tc_b_body,
        grid=(grid,),
        in_specs=[
            pl.BlockSpec((2, blk, ACC1_W), lambda i: (0, i, 0)),
            pl.BlockSpec((1, 128), lambda i: (0, 0)),
            pl.BlockSpec((128, D_OUT), lambda i: (0, 0)),
            pl.BlockSpec((8, 128), lambda i: (0, 0)),
            pl.BlockSpec((D_OUT, 16), lambda i: (0, 0)),
            pl.BlockSpec((D_OUT, 16), lambda i: (0, 0)),
        ],
        out_specs=[
            pl.BlockSpec((blk, D_OUT), lambda i: (i, 0)),
            pl.BlockSpec((blk, 16), lambda i: (i, 0)),
            pl.BlockSpec((blk, 16), lambda i: (i, 0)),
        ],
        out_shape=[
            jax.ShapeDtypeStruct((NPAD, D_OUT), F32),
            jax.ShapeDtypeStruct((NPAD, 16), F32),
            jax.ShapeDtypeStruct((NPAD, 16), F32),
        ],
    )(parts1, b1.reshape(1, 128), W2, gexp, a2s, a2d)

    parts2 = _sc_layer2(sidx, didx, wv, as2, ad2, h2)

    out = pl.pallas_call(
        _tc_c_body,
        grid=(grid,),
        in_specs=[
            pl.BlockSpec((2, blk, ACC2_W), lambda i: (0, i, 0)),
            pl.BlockSpec((1, D_OUT), lambda i: (0, 0)),
        ],
        out_specs=pl.BlockSpec((blk, D_OUT), lambda i: (i, 0)),
        out_shape=jax.ShapeDtypeStruct((NPAD, D_OUT), F32),
    )(parts2, b2.reshape(1, D_OUT))

    return out[:N]


# speedup vs baseline: 39.4352x; 1.4154x over previous
import functools

import jax
import jax.numpy as jnp
from jax import lax
from jax.experimental import pallas as pl
from jax.experimental.pallas import tpu as pltpu
from jax.experimental.pallas import tpu_sc as plsc

N = 10000
E = 320000
D_IN = 128
HID = 16
HEADS = 8
D_OUT = 64

NPAD = 10240
NW = 32
K = 64
NCHUNK = 160
EW = NCHUNK * K
EPAD = NW * EW
ROWS_PER_TILE = NPAD // 16

ACC1_W = 136
ACC2_W = 80

F32 = jnp.float32
I32 = jnp.int32


def _tc_a_body(x_ref, w1_ref, af_ref, df_ref, gdup_ref, h_ref, as_ref, ad_ref):
    x = x_ref[...]
    h = jnp.dot(x, w1_ref[...], preferred_element_type=F32)
    h_ref[...] = h
    gdup = gdup_ref[...]
    as_ref[...] = jnp.dot(h * af_ref[...], gdup, preferred_element_type=F32)
    ad_ref[...] = jnp.dot(h * df_ref[...], gdup, preferred_element_type=F32)


def _tc_b_body(parts_ref, b1_ref, w2_ref, gexp_ref, a2s_ref, a2d_ref,
               h2_ref, as2_ref, ad2_ref):
    p = parts_ref[0] + parts_ref[1]
    num = p[:, 0:128]
    dend = p[:, 128:136]
    den = jnp.dot(dend, gexp_ref[...], preferred_element_type=F32)
    hcur = jnp.maximum(num / (den + 1e-16) + b1_ref[...], 0.0)
    h2 = jnp.dot(hcur, w2_ref[...], preferred_element_type=F32)
    h2_ref[...] = h2
    as2_ref[...] = jnp.dot(h2, a2s_ref[...], preferred_element_type=F32)
    ad2_ref[...] = jnp.dot(h2, a2d_ref[...], preferred_element_type=F32)


def _tc_c_body(parts_ref, b2_ref, out_ref):
    p = parts_ref[0] + parts_ref[1]
    num = p[:, 0:64]
    den = p[:, 64:65]
    z = num / (den + 1e-16) + b2_ref[...]
    m = jnp.max(z, axis=1, keepdims=True)
    lse = jnp.log(jnp.sum(jnp.exp(z - m), axis=1, keepdims=True)) + m
    out_ref[...] = z - lse


_MESH = plsc.VectorSubcoreMesh(core_axis_name="c", subcore_axis_name="s")

_GDN = lax.GatherDimensionNumbers(
    offset_dims=(), collapsed_slice_dims=(0,), start_index_map=(0,))


def _lane_bcast(v, h):
    idx = jnp.full((16,), h, I32)
    return lax.gather(v, idx[:, None], _GDN, (1,),
                      mode=lax.GatherScatterMode.PROMISE_IN_BOUNDS)


def _make_sc_kernel(feat_w, acc_w, n_head_vec):
    zvecs = acc_w // 16

    @functools.partial(
        pl.kernel,
        out_type=jax.ShapeDtypeStruct((2, NPAD, acc_w), F32),
        mesh=_MESH,
        compiler_params=pltpu.CompilerParams(
            needs_layout_passes=False, use_tc_tiling_on_sc=False),
        scratch_types=[
            pltpu.VMEM((3, K), I32),
            pltpu.VMEM((3, K), I32),
            pltpu.VMEM((K, 16), F32),
            pltpu.VMEM((K, 16), F32),
            pltpu.VMEM((K, 16), F32),
            pltpu.VMEM((K, 16), F32),
            pltpu.VMEM((K, feat_w), F32),
            pltpu.VMEM((K, feat_w), F32),
            pltpu.VMEM((K, acc_w), F32),
            pltpu.VMEM((K, acc_w), F32),
            pltpu.VMEM_SHARED((NPAD, acc_w), F32),
            pltpu.SemaphoreType.DMA,
            pltpu.SemaphoreType.DMA,
            pltpu.SemaphoreType.DMA,
            pltpu.SemaphoreType.DMA,
        ],
    )
    def sc_kernel(edata, asrc, adst, htab, out,
                  ed0, ed1, ar0, ar1, br0, br1, hr0, hr1, m0, m1, acc,
                  se0, se1, sg0, sg1):
        cid = lax.axis_index("c")
        sid = lax.axis_index("s")
        wid = cid * 16 + sid
        zero16 = jnp.zeros((16,), F32)
        bufs = ((ed0, ar0, br0, hr0, m0, se0, sg0),
                (ed1, ar1, br1, hr1, m1, se1, sg1))

        @pl.loop(0, K)
        def _zero_msg(j):
            for t in range(zvecs):
                m0[j, pl.ds(16 * t, 16)] = zero16

        for b in range(ROWS_PER_TILE // K):
            pltpu.sync_copy(m0, acc.at[pl.ds(sid * ROWS_PER_TILE + b * K, K)])
        plsc.subcore_barrier()

        tbase = wid * NCHUNK

        def issue_gathers(ed, ar, br, hr, sg):
            pltpu.async_copy(asrc.at[ed.at[0]], ar, sg)
            pltpu.async_copy(adst.at[ed.at[1]], br, sg)
            pltpu.async_copy(htab.at[ed.at[0]], hr, sg)

        def wait_gathers(ed, ar, br, hr, sg):
            pltpu.make_async_copy(asrc.at[ed.at[0]], ar, sg).wait()
            pltpu.make_async_copy(adst.at[ed.at[1]], br, sg).wait()
            pltpu.make_async_copy(htab.at[ed.at[0]], hr, sg).wait()

        pltpu.async_copy(edata.at[tbase], ed0, se0).wait()
        issue_gathers(ed0, ar0, br0, hr0, sg0)
        pltpu.async_copy(edata.at[tbase + 1], ed1, se1)

        @pl.loop(0, NCHUNK // 2)
        def _pair(g):
            for u in (0, 1):
                cc = 2 * g + u
                ed, ar, br, hr, m, se, sg = bufs[u]
                edn, arn, brn, hrn, mn, sen, sgn = bufs[1 - u]

                @pl.when(cc + 1 < NCHUNK)
                def _prefetch_next():
                    pltpu.make_async_copy(
                        edata.at[tbase + cc + 1], edn, sen).wait()
                    issue_gathers(edn, arn, brn, hrn, sgn)

                wait_gathers(ed, ar, br, hr, sg)

                @pl.loop(0, K)
                def _edge(j):
                    va = ar[j, pl.ds(0, 16)]
                    vb = br[j, pl.ds(0, 16)]
                    e = va + vb
                    e = jnp.where(e >= 0.0, e, 0.2 * e)
                    ex = jnp.exp(e)
                    wb = plsc.load_gather(
                        ed, [jnp.full((16,), 2, I32), jnp.full((16,), j, I32)])
                    cf = ex * plsc.bitcast(wb, F32)
                    if n_head_vec:
                        m[j, pl.ds(feat_w - 8, 16)] = ex
                        for h in range(n_head_vec):
                            hv = hr[j, pl.ds(16 * h, 16)]
                            m[j, pl.ds(16 * h, 16)] = hv * _lane_bcast(cf, h)
                    else:
                        m[j, pl.ds(feat_w, 16)] = ex
                        for t in range(feat_w // 16):
                            hv = hr[j, pl.ds(16 * t, 16)]
                            m[j, pl.ds(16 * t, 16)] = hv * cf

                pltpu.sync_copy(m, acc.at[ed.at[1]], add=True)

                @pl.when(cc + 2 < NCHUNK)
                def _prefetch_records():
                    pltpu.async_copy(edata.at[tbase + cc + 2], ed, se)

        plsc.subcore_barrier()

        pltpu.sync_copy(
            acc.at[pl.ds(sid * ROWS_PER_TILE, ROWS_PER_TILE)],
            out.at[cid, pl.ds(sid * ROWS_PER_TILE, ROWS_PER_TILE)])

    return sc_kernel


_sc_layer1 = _make_sc_kernel(128, ACC1_W, 8)
_sc_layer2 = _make_sc_kernel(64, ACC2_W, 0)


@jax.jit
def kernel(X, A, W, W1, a_src1, a_dst1, b1, W2, a_src2, a_dst2, b2):
    xp = jnp.zeros((NPAD, D_IN), F32).at[:N].set(X)
    pad_idx = jnp.full((EPAD - E,), NPAD - 1, I32)
    sp = jnp.concatenate([A[0], pad_idx]).reshape(NW, NCHUNK, K)
    dp = jnp.concatenate([A[1], pad_idx]).reshape(NW, NCHUNK, K)
    wb = lax.bitcast_convert_type(
        jnp.concatenate([W, jnp.zeros((EPAD - E,), F32)]), I32
    ).reshape(NW, NCHUNK, K)
    edata = jnp.stack([sp, dp, wb], axis=2).reshape(NW * NCHUNK, 3, K)

    lane = jnp.arange(128)
    head16 = jnp.arange(16)
    gdup = (lane[:, None] // 16 == (head16[None, :] % 8)).astype(F32)
    head8 = jnp.arange(8)
    gexp = (lane[None, :] // 16 == head8[:, None]).astype(F32)
    af = a_src1.reshape(1, 128)
    df = a_dst1.reshape(1, 128)
    a2s = jnp.tile(a_src2.reshape(D_OUT, 1), (1, 16))
    a2d = jnp.tile(a_dst2.reshape(D_OUT, 1), (1, 16))

    blk = 1024
    grid = NPAD // blk

    h1, asd, add_ = pl.pallas_call(
        _tc_a_body,
        grid=(grid,),
        in_specs=[
            pl.BlockSpec((blk, D_IN), lambda i: (i, 0)),
            pl.BlockSpec((D_IN, 128), lambda i: (0, 0)),
            pl.BlockSpec((1, 128), lambda i: (0, 0)),
            pl.BlockSpec((1, 128), lambda i: (0, 0)),
            pl.BlockSpec((128, 16), lambda i: (0, 0)),
        ],
        out_specs=[
            pl.BlockSpec((blk, 128), lambda i: (i, 0)),
            pl.BlockSpec((blk, 16), lambda i: (i, 0)),
            pl.BlockSpec((blk, 16), lambda i: (i, 0)),
        ],
        out_shape=[
            jax.ShapeDtypeStruct((NPAD, 128), F32),
            jax.ShapeDtypeStruct((NPAD, 16), F32),
            jax.ShapeDtypeStruct((NPAD, 16), F32),
        ],
    )(xp, W1, af, df, gdup)

    parts1 = _sc_layer1(edata, asd, add_, h1)

    h2, as2, ad2 = pl.pallas_call(
        _tc_b_body,
        grid=(grid,),
        in_specs=[
            pl.BlockSpec((2, blk, ACC1_W), lambda i: (0, i, 0)),
            pl.BlockSpec((1, 128), lambda i: (0, 0)),
            pl.BlockSpec((128, D_OUT), lambda i: (0, 0)),
            pl.BlockSpec((8, 128), lambda i: (0, 0)),
            pl.BlockSpec((D_OUT, 16), lambda i: (0, 0)),
            pl.BlockSpec((D_OUT, 16), lambda i: (0, 0)),
        ],
        out_specs=[
            pl.BlockSpec((blk, D_OUT), lambda i: (i, 0)),
            pl.BlockSpec((blk, 16), lambda i: (i, 0)),
            pl.BlockSpec((blk, 16), lambda i: (i, 0)),
        ],
        out_shape=[
            jax.ShapeDtypeStruct((NPAD, D_OUT), F32),
            jax.ShapeDtypeStruct((NPAD, 16), F32),
            jax.ShapeDtypeStruct((NPAD, 16), F32),
        ],
    )(parts1, b1.reshape(1, 128), W2, gexp, a2s, a2d)

    parts2 = _sc_layer2(edata, as2, ad2, h2)

    out = pl.pallas_call(
        _tc_c_body,
        grid=(grid,),
        in_specs=[
            pl.BlockSpec((2, blk, ACC2_W), lambda i: (0, i, 0)),
            pl.BlockSpec((1, D_OUT), lambda i: (0, 0)),
        ],
        out_specs=pl.BlockSpec((blk, D_OUT), lambda i: (i, 0)),
        out_shape=jax.ShapeDtypeStruct((NPAD, D_OUT), F32),
    )(parts2, b2.reshape(1, D_OUT))

    return out[:N]
